# baseline (speedup 1.0000x reference)
import jax
import jax.numpy as jnp
from jax.experimental import pallas as pl
from jax.experimental.pallas import tpu as pltpu

N = 1024
IN_F = 256
OUT_F = 64
ALPHA = 0.2
BR = 128
K = N // BR


def _gat_body(x_ref, adj_ref, w_ref, a1_ref, a2_ref, bias_ref, wrt_ref,
              gamma_ref, beta_ref, out_ref,
              haug_s, f_s, g_s, m_s, a_s, res_s):
    i = pl.program_id(0)

    @pl.when(i == 0)
    def _prologue():
        x = x_ref[...]
        h = jnp.dot(x, w_ref[...], preferred_element_type=jnp.float32) \
            + bias_ref[...]
        haug_s[...] = jnp.concatenate(
            [h, jnp.ones((N, 1), jnp.float32)], axis=1)
        f_s[...] = jnp.sum(h * a1_ref[...], axis=1, keepdims=True)
        g_s[...] = jnp.sum(h * a2_ref[...], axis=1, keepdims=True).T
        res_s[...] = jnp.dot(x, wrt_ref[...],
                             preferred_element_type=jnp.float32)

    s = f_s[pl.ds(i * BR, BR), :] + g_s[...]
    s = jnp.maximum(s, ALPHA * s)
    mask = adj_ref[...] != 0
    m_i = jnp.max(jnp.where(mask, s, -jnp.inf), axis=1, keepdims=True)
    e = jnp.where(mask, jnp.exp(s - m_i), 0.0)
    a_s[pl.ds(i * BR, BR), :] = jnp.dot(
        e, haug_s[...], preferred_element_type=jnp.float32)
    m_s[pl.ds(i * BR, BR), :] = m_i

    @pl.when(i == K - 1)
    def _epilogue():
        m_all = m_s[...]
        m_glob = jnp.max(m_all)
        corr = 1e-8 * jnp.exp(m_glob - m_all)
        aaug = a_s[...]
        hp = aaug[:, :OUT_F] / (aaug[:, OUT_F:] + corr) + res_s[...]
        mean = jnp.mean(hp, axis=-1, keepdims=True)
        c = hp - mean
        var = jnp.mean(c * c, axis=-1, keepdims=True)
        hn = c * jax.lax.rsqrt(var + 1e-5) * gamma_ref[...] + beta_ref[...]
        out_ref[...] = jnp.where(hn > 0, hn,
                                 jnp.exp(jnp.minimum(hn, 0.0)) - 1.0)


def kernel(input, adj, W, a_param, bias, W_res, ln_gamma, ln_beta):
    a1 = a_param[:, :OUT_F].reshape(1, OUT_F)
    a2 = a_param[:, OUT_F:].reshape(1, OUT_F)
    full = lambda r, c: pl.BlockSpec((r, c), lambda i: (0, 0))
    return pl.pallas_call(
        _gat_body,
        grid=(K,),
        in_specs=[
            full(N, IN_F),
            pl.BlockSpec((BR, N), lambda i: (i, 0)),
            full(IN_F, OUT_F),
            full(1, OUT_F), full(1, OUT_F),
            full(1, OUT_F),
            full(IN_F, OUT_F),
            full(1, OUT_F), full(1, OUT_F),
        ],
        out_specs=full(N, OUT_F),
        out_shape=jax.ShapeDtypeStruct((N, OUT_F), jnp.float32),
        scratch_shapes=[
            pltpu.VMEM((N, OUT_F + 1), jnp.float32),
            pltpu.VMEM((N, 1), jnp.float32),
            pltpu.VMEM((1, N), jnp.float32),
            pltpu.VMEM((N, 1), jnp.float32),
            pltpu.VMEM((N, OUT_F + 1), jnp.float32),
            pltpu.VMEM((N, OUT_F), jnp.float32),
        ],
        compiler_params=pltpu.CompilerParams(
            dimension_semantics=("arbitrary",)),
    )(input, adj, W, a1, a2, bias.reshape(1, OUT_F), W_res.T,
      ln_gamma.reshape(1, OUT_F), ln_beta.reshape(1, OUT_F))


# speedup vs baseline: 1.1477x; 1.1477x over previous
import jax
import jax.numpy as jnp
from jax.experimental import pallas as pl

N = 1024
OUT_F = 64
ALPHA = 0.2


def _gat_body(x_ref, adj_ref, w_ref, a1_ref, a2_ref, bias_ref, wrt_ref,
              gamma_ref, beta_ref, out_ref):
    x = x_ref[...]

    h = jnp.dot(x, w_ref[...], preferred_element_type=jnp.float32) \
        + bias_ref[...]

    f = jnp.sum(h * a1_ref[...], axis=1, keepdims=True)
    g = jnp.sum(h * a2_ref[...], axis=1, keepdims=True)

    t = f + jnp.max(g)
    c = jnp.maximum(t, ALPHA * t)

    s = f + g.T
    s = jnp.maximum(s, ALPHA * s)
    e = jnp.where(adj_ref[...] != 0, jnp.exp(s - c), 0.0)

    rmax = jnp.max(e, axis=1, keepdims=True)
    m = jnp.max(c + jnp.log(rmax))
    corr = 1e-8 * jnp.exp(m - c)

    haug = jnp.concatenate([h, jnp.ones((N, 1), jnp.float32)], axis=1)
    aaug = jnp.dot(e, haug, preferred_element_type=jnp.float32)

    hp = aaug[:, :OUT_F] / (aaug[:, OUT_F:] + corr)
    hp = hp + jnp.dot(x, wrt_ref[...], preferred_element_type=jnp.float32)

    mean = jnp.mean(hp, axis=-1, keepdims=True)
    cen = hp - mean
    var = jnp.mean(cen * cen, axis=-1, keepdims=True)
    hn = cen * jax.lax.rsqrt(var + 1e-5) * gamma_ref[...] + beta_ref[...]

    out_ref[...] = jnp.where(hn > 0, hn, jnp.exp(jnp.minimum(hn, 0.0)) - 1.0)


def kernel(input, adj, W, a_param, bias, W_res, ln_gamma, ln_beta):
    a1 = a_param[:, :OUT_F].reshape(1, OUT_F)
    a2 = a_param[:, OUT_F:].reshape(1, OUT_F)
    return pl.pallas_call(
        _gat_body,
        out_shape=jax.ShapeDtypeStruct((N, OUT_F), jnp.float32),
    )(input, adj, W, a1, a2, bias.reshape(1, OUT_F), W_res.T,
      ln_gamma.reshape(1, OUT_F), ln_beta.reshape(1, OUT_F))


# speedup vs baseline: 1.5407x; 1.3425x over previous
import jax
import jax.numpy as jnp
from jax.experimental import pallas as pl

N = 1024
OUT_F = 64
ALPHA = 0.2


def _gat_body(x_ref, adj_ref, w_ref, ap_ref, bias_ref, wres_ref,
              gamma_ref, beta_ref, out_ref):
    x = x_ref[...]
    adj = adj_ref[...]

    h = jnp.dot(x, w_ref[...], preferred_element_type=jnp.float32) \
        + bias_ref[...].reshape(1, OUT_F)

    a1 = ap_ref[:, :OUT_F]
    a2 = ap_ref[:, OUT_F:]
    f = jnp.sum(h * a1, axis=1, keepdims=True)
    g = jnp.sum(h * a2, axis=1, keepdims=True)
    s = f + g.T
    s = jnp.where(s >= 0, s, ALPHA * s)

    mask = adj != 0
    m = jnp.max(jnp.where(mask, s, -jnp.inf))
    e = jnp.where(mask, jnp.exp(s - m), 0.0)

    rowsum = jnp.sum(e, axis=1, keepdims=True) + 1e-8
    hp = jnp.dot(e, h, preferred_element_type=jnp.float32) / rowsum

    res = jax.lax.dot_general(x, wres_ref[...],
                              (((1,), (1,)), ((), ())),
                              preferred_element_type=jnp.float32)
    hp = hp + res

    mean = jnp.mean(hp, axis=-1, keepdims=True)
    c = hp - mean
    var = jnp.mean(c * c, axis=-1, keepdims=True)
    hn = c * jax.lax.rsqrt(var + 1e-5) \
        * gamma_ref[...].reshape(1, OUT_F) \
        + beta_ref[...].reshape(1, OUT_F)

    out_ref[...] = jnp.where(hn > 0, hn, jnp.exp(jnp.minimum(hn, 0.0)) - 1.0)


def kernel(input, adj, W, a_param, bias, W_res, ln_gamma, ln_beta):
    return pl.pallas_call(
        _gat_body,
        out_shape=jax.ShapeDtypeStruct((N, OUT_F), jnp.float32),
    )(input, adj, W, a_param, bias, W_res, ln_gamma, ln_beta)
